# baseline (speedup 1.0000x reference)
import functools

import jax
import jax.numpy as jnp
from jax import lax
from jax.experimental import pallas as pl
from jax.experimental.pallas import tpu as pltpu
from jax.experimental.pallas import tpu_sc as plsc

_ROWS, _COLS = 160, 128
_BLK = 8


@functools.lru_cache(maxsize=None)
def _make_lookup(table_n: int, num_subcores: int):
    n_blocks = _ROWS // _BLK
    mesh = plsc.VectorSubcoreMesh(
        core_axis_name="c", subcore_axis_name="s", num_cores=1)

    @functools.partial(
        pl.kernel,
        mesh=mesh,
        out_type=jax.ShapeDtypeStruct((_ROWS, _COLS), jnp.float32),
        scratch_types=[
            pltpu.VMEM_SHARED((table_n,), jnp.float32),
            pltpu.VMEM((_BLK, _COLS), jnp.int32),
            pltpu.VMEM((_BLK, _COLS), jnp.float32),
            pltpu.SemaphoreType.DMA,
            pltpu.SemaphoreType.DMA,
        ],
    )
    def lookup(ents_hbm, table_hbm, out_hbm, table_sh, idx_v, out_v, sem,
               stage_sem):
        sid = lax.axis_index("s")

        @pl.when(sid == 0)
        def _start_table_stage():
            pltpu.async_copy(table_hbm, table_sh, stage_sem)

        @pl.when(sid == 0)
        def _finish_table_stage():
            pltpu.make_async_copy(table_hbm, table_sh, stage_sem).wait()

        plsc.subcore_barrier()

        def do_block(blk):
            base = blk * _BLK
            pltpu.sync_copy(ents_hbm.at[pl.ds(base, _BLK)], idx_v)

            def row(i, carry):
                pltpu.async_copy(table_sh.at[idx_v.at[i]], out_v.at[i],
                                 sem).wait()
                return carry

            lax.fori_loop(0, _BLK, row, 0)
            pltpu.sync_copy(out_v, out_hbm.at[pl.ds(base, _BLK)])

        do_block(sid)

        @pl.when(sid < n_blocks - num_subcores)
        def _second_block():
            do_block(sid + num_subcores)

    return lookup


def kernel(entities, ent2cluster_key, ent2cluster_value):
    del ent2cluster_key
    shape = entities.shape
    ents = entities.reshape(_ROWS, _COLS).astype(jnp.int32)
    table = ent2cluster_value.astype(jnp.float32)
    info = plsc.get_sparse_core_info()
    out = _make_lookup(table.shape[0], info.num_subcores)(ents, table)
    return out.reshape(shape)


# speedup vs baseline: 1.1544x; 1.1544x over previous
import functools

import jax
import jax.numpy as jnp
from jax import lax
from jax.experimental import pallas as pl
from jax.experimental.pallas import tpu as pltpu
from jax.experimental.pallas import tpu_sc as plsc


@functools.lru_cache(maxsize=None)
def _make_lookup(n_flat: int, table_n: int, num_cores: int,
                 num_subcores: int):
    num_workers = num_cores * num_subcores
    chunk = n_flat // num_workers
    assert chunk * num_workers == n_flat and chunk % 8 == 0
    mesh = plsc.VectorSubcoreMesh(
        core_axis_name="c", subcore_axis_name="s", num_cores=num_cores)

    @functools.partial(
        pl.kernel,
        mesh=mesh,
        out_type=jax.ShapeDtypeStruct((n_flat,), jnp.float32),
        scratch_types=[
            pltpu.VMEM_SHARED((table_n,), jnp.float32),
            pltpu.VMEM((chunk,), jnp.int32),
            pltpu.VMEM((chunk,), jnp.float32),
            pltpu.SemaphoreType.DMA,
            pltpu.SemaphoreType.DMA,
        ],
    )
    def lookup(ents_hbm, table_hbm, out_hbm, table_sh, idx_v, out_v, sem,
               stage_sem):
        cid = lax.axis_index("c")
        sid = lax.axis_index("s")
        wid = sid * num_cores + cid
        base = wid * chunk

        @pl.when(sid == 0)
        def _start_table_stage():
            pltpu.async_copy(table_hbm, table_sh, stage_sem)

        pltpu.sync_copy(ents_hbm.at[pl.ds(base, chunk)], idx_v)

        @pl.when(sid == 0)
        def _finish_table_stage():
            pltpu.make_async_copy(table_hbm, table_sh, stage_sem).wait()

        plsc.subcore_barrier()
        pltpu.async_copy(table_sh.at[idx_v], out_v, sem).wait()
        pltpu.sync_copy(out_v, out_hbm.at[pl.ds(base, chunk)])

    return lookup


def kernel(entities, ent2cluster_key, ent2cluster_value):
    del ent2cluster_key
    shape = entities.shape
    flat = entities.reshape(-1).astype(jnp.int32)
    table = ent2cluster_value.astype(jnp.float32)
    info = plsc.get_sparse_core_info()
    out = _make_lookup(flat.shape[0], table.shape[0], 1, info.num_subcores)(
        flat, table)
    return out.reshape(shape)
